# baseline (speedup 1.0000x reference)
import functools

import jax
import jax.numpy as jnp
from jax import lax
from jax.experimental import pallas as pl
from jax.experimental.pallas import tpu as pltpu
from jax.experimental.pallas import tpu_sc as plsc

NUM_PID = 5532
THRESHOLD = 0.2
LUT_DIM = 256
BATCH = 16384

A_BLK = 512
A_GRID = (NUM_PID + A_BLK - 1) // A_BLK
NP_PAD = A_GRID * A_BLK

B_BLK = 512
B_NSTREAM = 2
B_GRID = BATCH // (B_BLK * B_NSTREAM)
B_HALF_BLOCKS = BATCH // (B_BLK * B_NSTREAM)

SC_CORES = 2
SC_SUBCORES = 16
SC_LANES = 16
SC_WORKERS = SC_CORES * SC_SUBCORES
B_PER_W = BATCH // SC_WORKERS


def _simmax_kernel(lut_blk_ref, lut_full_ref, maxval_ref, maxind_ref):
    i = pl.program_id(0)
    sim = jax.lax.dot_general(
        lut_blk_ref[...], lut_full_ref[...],
        (((1,), (1,)), ((), ())),
        preferred_element_type=jnp.float32,
    )
    row = i * A_BLK + lax.broadcasted_iota(jnp.int32, (A_BLK, NUM_PID), 0)
    col = lax.broadcasted_iota(jnp.int32, (A_BLK, NUM_PID), 1)
    sim = jnp.where(col == row, 0.0, sim)
    m = jnp.max(sim, axis=1)
    ind = jnp.min(jnp.where(sim == m[:, None], col, NUM_PID), axis=1)
    maxval_ref[...] = m
    maxind_ref[...] = ind


def _sim_max_argmax(lut):
    return pl.pallas_call(
        _simmax_kernel,
        grid=(A_GRID,),
        in_specs=[
            pl.BlockSpec((A_BLK, LUT_DIM), lambda i: (i, 0)),
            pl.BlockSpec((NUM_PID, LUT_DIM), lambda i: (0, 0)),
        ],
        out_specs=[
            pl.BlockSpec((A_BLK,), lambda i: (i,)),
            pl.BlockSpec((A_BLK,), lambda i: (i,)),
        ],
        out_shape=[
            jax.ShapeDtypeStruct((NP_PAD,), jnp.float32),
            jax.ShapeDtypeStruct((NP_PAD,), jnp.int32),
        ],
    )(lut.astype(jnp.bfloat16), lut.astype(jnp.bfloat16))


def _killcol_sc_kernel(label_hbm, maxval_hbm, maxind_hbm, kill_hbm,
                       label_v, maxval_v, maxind_v, kill_v):
    wid = lax.axis_index("s") * SC_CORES + lax.axis_index("c")
    base = wid * B_PER_W
    pltpu.sync_copy(label_hbm.at[pl.ds(base, B_PER_W)], label_v)
    pltpu.sync_copy(maxval_hbm, maxval_v)
    pltpu.sync_copy(maxind_hbm, maxind_v)

    def body(j, _):
        off = j * SC_LANES
        lbl = label_v[pl.ds(off, SC_LANES)]
        g = jnp.minimum(lbl, NUM_PID - 1)
        val = plsc.load_gather(maxval_v, [g])
        ind = plsc.load_gather(maxind_v, [g])
        ignore = (lbl < NUM_PID) & (val > THRESHOLD)
        kill_v[pl.ds(off, SC_LANES)] = jnp.where(ignore, ind, -1)
        return 0

    lax.fori_loop(0, B_PER_W // SC_LANES, body, 0)
    pltpu.sync_copy(kill_v, kill_hbm.at[base // B_BLK, 0, pl.ds(base % B_BLK, B_PER_W)])


@functools.partial(jax.jit, static_argnames=())
def _kill_cols(label, maxval, maxind):
    mesh = plsc.VectorSubcoreMesh(core_axis_name="c", subcore_axis_name="s")
    k = pl.kernel(
        _killcol_sc_kernel,
        mesh=mesh,
        out_type=jax.ShapeDtypeStruct((BATCH // B_BLK, 1, B_BLK), jnp.int32),
        scratch_types=[
            pltpu.VMEM((B_PER_W,), jnp.int32),
            pltpu.VMEM((NP_PAD,), jnp.float32),
            pltpu.VMEM((NP_PAD,), jnp.int32),
            pltpu.VMEM((B_PER_W,), jnp.int32),
        ],
        compiler_params=pltpu.CompilerParams(needs_layout_passes=False),
    )
    return k(label, maxval, maxind)


def _stream_part(x, lbl, kill):
    col = lax.broadcasted_iota(jnp.int32, (B_BLK, NUM_PID), 1)
    xm = jnp.where(col == kill[:, None], 0.0, x)
    m = jnp.max(xm, axis=1)
    s = jnp.sum(jnp.exp(xm - m[:, None]), axis=1)
    g = jnp.minimum(lbl, NUM_PID - 1)
    picked = jnp.sum(jnp.where(col == g[:, None], xm, 0.0), axis=1)
    valid = (lbl < NUM_PID).astype(jnp.float32)
    per_row = (picked - m - jnp.log(s)) * valid
    return per_row.reshape(B_BLK // 128, 128), valid.reshape(B_BLK // 128, 128)


def _loss_kernel(logits0_ref, logits1_ref, label0_ref, label1_ref,
                 kill0_ref, kill1_ref, out_ref, loss_ref):
    i = pl.program_id(0)
    p0, c0 = _stream_part(logits0_ref[0], label0_ref[...],
                          kill0_ref[0, 0, :])
    p1, c1 = _stream_part(logits1_ref[0], label1_ref[...],
                          kill1_ref[0, 0, :])

    @pl.when(i == 0)
    def _():
        out_ref[...] = jnp.zeros_like(out_ref)

    h = B_BLK // 128
    out_ref[0:h, :] += p0 + p1
    out_ref[h:2 * h, :] += c0 + c1

    @pl.when(i == B_GRID - 1)
    def _():
        num = jnp.sum(out_ref[0:h, :])
        cnt = jnp.maximum(jnp.sum(out_ref[h:2 * h, :]), 1.0)
        loss_ref[...] = jnp.broadcast_to(-num / cnt, (8, 128))


def _stream_loss(logits, label, kill):
    kill3 = kill
    lg = logits.reshape(B_NSTREAM, BATCH // B_NSTREAM, NUM_PID)
    _, loss = pl.pallas_call(
        _loss_kernel,
        grid=(B_GRID,),
        in_specs=[
            pl.BlockSpec((1, B_BLK, NUM_PID), lambda i: (0, i, 0)),
            pl.BlockSpec((1, B_BLK, NUM_PID), lambda i: (1, i, 0)),
            pl.BlockSpec((B_BLK,), lambda i: (i,)),
            pl.BlockSpec((B_BLK,), lambda i: (i + B_GRID,)),
            pl.BlockSpec((1, 1, B_BLK), lambda i: (i, 0, 0)),
            pl.BlockSpec((1, 1, B_BLK), lambda i: (i + B_GRID, 0, 0)),
        ],
        out_specs=[
            pl.BlockSpec((2 * B_BLK // 128, 128), lambda i: (0, 0)),
            pl.BlockSpec((8, 128), lambda i: (0, 0)),
        ],
        out_shape=[
            jax.ShapeDtypeStruct((2 * B_BLK // 128, 128), jnp.float32),
            jax.ShapeDtypeStruct((8, 128), jnp.float32),
        ],
        compiler_params=pltpu.CompilerParams(
            vmem_limit_bytes=62 * 1024 * 1024),
    )(lg, lg, label, label, kill3, kill3)
    return loss


def kernel(logits, label, lut):
    label = label.astype(jnp.int32)
    maxval, maxind = _sim_max_argmax(lut)
    kill = _kill_cols(label, maxval, maxind)
    loss = _stream_loss(logits, label, kill)
    return loss[0, 0]


# speedup vs baseline: 1.0061x; 1.0061x over previous
import functools

import jax
import jax.numpy as jnp
from jax import lax
from jax.experimental import pallas as pl
from jax.experimental.pallas import tpu as pltpu
from jax.experimental.pallas import tpu_sc as plsc

NUM_PID = 5532
THRESHOLD = 0.2
LUT_DIM = 256
BATCH = 16384

A_BLK = 1024
A_GRID = (NUM_PID + A_BLK - 1) // A_BLK
NP_PAD = A_GRID * A_BLK

B_BLK = 512
B_NSTREAM = 2
B_GRID = BATCH // (B_BLK * B_NSTREAM)
B_HALF_BLOCKS = BATCH // (B_BLK * B_NSTREAM)

SC_CORES = 2
SC_SUBCORES = 16
SC_LANES = 16
SC_WORKERS = SC_CORES * SC_SUBCORES
B_PER_W = BATCH // SC_WORKERS


def _simmax_kernel(lut_blk_ref, lut_full_ref, maxval_ref, maxind_ref):
    i = pl.program_id(0)
    sim = jax.lax.dot_general(
        lut_blk_ref[...], lut_full_ref[...],
        (((1,), (1,)), ((), ())),
        preferred_element_type=jnp.float32,
    )
    row = i * A_BLK + lax.broadcasted_iota(jnp.int32, (A_BLK, NUM_PID), 0)
    col = lax.broadcasted_iota(jnp.int32, (A_BLK, NUM_PID), 1)
    sim = jnp.where(col == row, 0.0, sim)
    m = jnp.max(sim, axis=1)
    ind = jnp.min(jnp.where(sim == m[:, None], col, NUM_PID), axis=1)
    maxval_ref[...] = m
    maxind_ref[...] = ind


def _sim_max_argmax(lut):
    return pl.pallas_call(
        _simmax_kernel,
        grid=(A_GRID,),
        in_specs=[
            pl.BlockSpec((A_BLK, LUT_DIM), lambda i: (i, 0)),
            pl.BlockSpec((NUM_PID, LUT_DIM), lambda i: (0, 0)),
        ],
        out_specs=[
            pl.BlockSpec((A_BLK,), lambda i: (i,)),
            pl.BlockSpec((A_BLK,), lambda i: (i,)),
        ],
        out_shape=[
            jax.ShapeDtypeStruct((NP_PAD,), jnp.float32),
            jax.ShapeDtypeStruct((NP_PAD,), jnp.int32),
        ],
    )(lut, lut)


def _killcol_sc_kernel(label_hbm, maxval_hbm, maxind_hbm, kill_hbm,
                       label_v, maxval_v, maxind_v, kill_v):
    wid = lax.axis_index("s") * SC_CORES + lax.axis_index("c")
    base = wid * B_PER_W
    pltpu.sync_copy(label_hbm.at[pl.ds(base, B_PER_W)], label_v)
    pltpu.sync_copy(maxval_hbm, maxval_v)
    pltpu.sync_copy(maxind_hbm, maxind_v)

    def body(j, _):
        off = j * SC_LANES
        lbl = label_v[pl.ds(off, SC_LANES)]
        g = jnp.minimum(lbl, NUM_PID - 1)
        val = plsc.load_gather(maxval_v, [g])
        ind = plsc.load_gather(maxind_v, [g])
        ignore = (lbl < NUM_PID) & (val > THRESHOLD)
        kill_v[pl.ds(off, SC_LANES)] = jnp.where(ignore, ind, -1)
        return 0

    lax.fori_loop(0, B_PER_W // SC_LANES, body, 0)
    pltpu.sync_copy(kill_v, kill_hbm.at[base // B_BLK, 0, pl.ds(base % B_BLK, B_PER_W)])


@functools.partial(jax.jit, static_argnames=())
def _kill_cols(label, maxval, maxind):
    mesh = plsc.VectorSubcoreMesh(core_axis_name="c", subcore_axis_name="s")
    k = pl.kernel(
        _killcol_sc_kernel,
        mesh=mesh,
        out_type=jax.ShapeDtypeStruct((BATCH // B_BLK, 1, B_BLK), jnp.int32),
        scratch_types=[
            pltpu.VMEM((B_PER_W,), jnp.int32),
            pltpu.VMEM((NP_PAD,), jnp.float32),
            pltpu.VMEM((NP_PAD,), jnp.int32),
            pltpu.VMEM((B_PER_W,), jnp.int32),
        ],
        compiler_params=pltpu.CompilerParams(needs_layout_passes=False),
    )
    return k(label, maxval, maxind)


def _stream_part(x, lbl, kill):
    col = lax.broadcasted_iota(jnp.int32, (B_BLK, NUM_PID), 1)
    xm = jnp.where(col == kill[:, None], 0.0, x)
    m = jnp.max(xm, axis=1)
    s = jnp.sum(jnp.exp(xm - m[:, None]), axis=1)
    g = jnp.minimum(lbl, NUM_PID - 1)
    picked = jnp.sum(jnp.where(col == g[:, None], xm, 0.0), axis=1)
    valid = (lbl < NUM_PID).astype(jnp.float32)
    per_row = (picked - m - jnp.log(s)) * valid
    return per_row.reshape(B_BLK // 128, 128), valid.reshape(B_BLK // 128, 128)


def _loss_kernel(logits0_ref, logits1_ref, label0_ref, label1_ref,
                 kill0_ref, kill1_ref, out_ref, loss_ref):
    i = pl.program_id(0)
    p0, c0 = _stream_part(logits0_ref[0], label0_ref[...],
                          kill0_ref[0, 0, :])
    p1, c1 = _stream_part(logits1_ref[0], label1_ref[...],
                          kill1_ref[0, 0, :])

    @pl.when(i == 0)
    def _():
        out_ref[...] = jnp.zeros_like(out_ref)

    h = B_BLK // 128
    out_ref[0:h, :] += p0 + p1
    out_ref[h:2 * h, :] += c0 + c1

    @pl.when(i == B_GRID - 1)
    def _():
        num = jnp.sum(out_ref[0:h, :])
        cnt = jnp.maximum(jnp.sum(out_ref[h:2 * h, :]), 1.0)
        loss_ref[...] = jnp.broadcast_to(-num / cnt, (8, 128))


def _stream_loss(logits, label, kill):
    kill3 = kill
    lg = logits.reshape(B_NSTREAM, BATCH // B_NSTREAM, NUM_PID)
    _, loss = pl.pallas_call(
        _loss_kernel,
        grid=(B_GRID,),
        in_specs=[
            pl.BlockSpec((1, B_BLK, NUM_PID), lambda i: (0, i, 0)),
            pl.BlockSpec((1, B_BLK, NUM_PID), lambda i: (1, i, 0)),
            pl.BlockSpec((B_BLK,), lambda i: (i,)),
            pl.BlockSpec((B_BLK,), lambda i: (i + B_GRID,)),
            pl.BlockSpec((1, 1, B_BLK), lambda i: (i, 0, 0)),
            pl.BlockSpec((1, 1, B_BLK), lambda i: (i + B_GRID, 0, 0)),
        ],
        out_specs=[
            pl.BlockSpec((2 * B_BLK // 128, 128), lambda i: (0, 0)),
            pl.BlockSpec((8, 128), lambda i: (0, 0)),
        ],
        out_shape=[
            jax.ShapeDtypeStruct((2 * B_BLK // 128, 128), jnp.float32),
            jax.ShapeDtypeStruct((8, 128), jnp.float32),
        ],
        compiler_params=pltpu.CompilerParams(
            vmem_limit_bytes=62 * 1024 * 1024),
    )(lg, lg, label, label, kill3, kill3)
    return loss


def kernel(logits, label, lut):
    label = label.astype(jnp.int32)
    maxval, maxind = _sim_max_argmax(lut)
    kill = _kill_cols(label, maxval, maxind)
    loss = _stream_loss(logits, label, kill)
    return loss[0, 0]
